# baseline (speedup 1.0000x reference)
import jax
import jax.numpy as jnp
from jax import lax
from jax.experimental import pallas as pl

C = 384
B, H, W = 16, 56, 56
HW = H * W
ROWS = B * C
ROW_BLK = 128
GRID = ROWS // ROW_BLK


def _mask_kernel(w_ref, mask_ref):
    w = jnp.abs(w_ref[...])
    mn = jnp.min(w, axis=1, keepdims=True)
    mx = jnp.max(w, axis=1, keepdims=True)
    step = lax.broadcasted_iota(jnp.int32, (1, C), 1).astype(jnp.float32) / jnp.float32(C)
    e = mn * (1.0 - step) + mx * step
    cmp = (w[:, :, None] >= e[:, None, :]).astype(jnp.float32)
    cnt = jnp.sum(cmp, axis=1)
    zcol = jnp.zeros((2, 1), jnp.float32)
    cnt_next = jnp.concatenate([cnt[:, 1:], zcol], axis=1)
    hist = cnt - cnt_next
    hist_next = jnp.concatenate([hist[:, 1:], zcol], axis=1)
    d = hist_next - hist
    d_next = jnp.concatenate([d[:, 1:], zcol], axis=1)
    cond = (d <= 0.0) & (d_next > 0.0)
    idx = lax.broadcasted_iota(jnp.int32, (1, C), 1).astype(jnp.float32)
    valid = idx <= jnp.float32(C - 3)
    cand = jnp.where(cond & valid, idx, jnp.float32(1e9))
    first = jnp.min(cand, axis=1, keepdims=True)
    i_star = jnp.where(first > jnp.float32(C - 2), 0.0, first)
    thr = mn + ((i_star + 2.0) * (mx - mn)) / jnp.float32(C)
    mask_ref[...] = (w >= thr).astype(jnp.float32)


def _exchange_kernel(m1_ref, m2_ref, x0_ref, x1_ref, o1_ref, o2_ref):
    m1 = m1_ref[...] != 0.0
    m2 = m2_ref[...] != 0.0
    x0 = x0_ref[...]
    x1 = x1_ref[...]
    o1_ref[...] = jnp.where(m1, x0, x1)
    o2_ref[...] = jnp.where(m2, x1, x0)


def kernel(x0, x1, bn1_weight, bn2_weight):
    wstack = jnp.stack([bn1_weight, bn2_weight])
    masks = pl.pallas_call(
        _mask_kernel,
        out_shape=jax.ShapeDtypeStruct((2, C), jnp.float32),
    )(wstack)
    m1 = jnp.broadcast_to(masks[0][None, :], (B, C)).reshape(ROWS, 1)
    m2 = jnp.broadcast_to(masks[1][None, :], (B, C)).reshape(ROWS, 1)

    x0v = x0.reshape(ROWS, HW)
    x1v = x1.reshape(ROWS, HW)

    row_spec = pl.BlockSpec((ROW_BLK, HW), lambda i: (i, 0))
    mask_spec = pl.BlockSpec((ROW_BLK, 1), lambda i: (i, 0))
    out1, out2 = pl.pallas_call(
        _exchange_kernel,
        grid=(GRID,),
        in_specs=[mask_spec, mask_spec, row_spec, row_spec],
        out_specs=[row_spec, row_spec],
        out_shape=[
            jax.ShapeDtypeStruct((ROWS, HW), jnp.float32),
            jax.ShapeDtypeStruct((ROWS, HW), jnp.float32),
        ],
    )(m1, m2, x0v, x1v)
    return (out1.reshape(B, C, H, W), out2.reshape(B, C, H, W))


# speedup vs baseline: 1.2409x; 1.2409x over previous
import jax
import jax.numpy as jnp
from jax import lax
from jax.experimental import pallas as pl

C = 384
B, H, W = 16, 56, 56
CB = 64
GRID_C = C // CB


def _mask_kernel(w_ref, mask_ref):
    w = jnp.abs(w_ref[...])
    mn = jnp.min(w, axis=1, keepdims=True)
    mx = jnp.max(w, axis=1, keepdims=True)
    step = lax.broadcasted_iota(jnp.int32, (1, C), 1).astype(jnp.float32) / jnp.float32(C)
    e = mn * (1.0 - step) + mx * step
    cmp = (w[:, :, None] >= e[:, None, :]).astype(jnp.float32)
    cnt = jnp.sum(cmp, axis=1)
    zcol = jnp.zeros((2, 1), jnp.float32)
    cnt_next = jnp.concatenate([cnt[:, 1:], zcol], axis=1)
    hist = cnt - cnt_next
    hist_next = jnp.concatenate([hist[:, 1:], zcol], axis=1)
    d = hist_next - hist
    d_next = jnp.concatenate([d[:, 1:], zcol], axis=1)
    cond = (d <= 0.0) & (d_next > 0.0)
    idx = lax.broadcasted_iota(jnp.int32, (1, C), 1).astype(jnp.float32)
    valid = idx <= jnp.float32(C - 3)
    cand = jnp.where(cond & valid, idx, jnp.float32(1e9))
    first = jnp.min(cand, axis=1, keepdims=True)
    i_star = jnp.where(first > jnp.float32(C - 2), 0.0, first)
    thr = mn + ((i_star + 2.0) * (mx - mn)) / jnp.float32(C)
    mask_ref[...] = (w >= thr).astype(jnp.float32)


def _exchange_kernel(m1_ref, m2_ref, x0_ref, x1_ref, o1_ref, o2_ref):
    m1 = m1_ref[...][None] != 0.0
    m2 = m2_ref[...][None] != 0.0
    x0 = x0_ref[...]
    x1 = x1_ref[...]
    o1_ref[...] = jnp.where(m1, x0, x1)
    o2_ref[...] = jnp.where(m2, x1, x0)


def kernel(x0, x1, bn1_weight, bn2_weight):
    wstack = jnp.stack([bn1_weight, bn2_weight])
    masks = pl.pallas_call(
        _mask_kernel,
        out_shape=jax.ShapeDtypeStruct((2, C), jnp.float32),
    )(wstack)
    m1 = masks[0].reshape(C, 1, 1)
    m2 = masks[1].reshape(C, 1, 1)

    x_spec = pl.BlockSpec((1, CB, H, W), lambda b, c: (b, c, 0, 0))
    mask_spec = pl.BlockSpec((CB, 1, 1), lambda b, c: (c, 0, 0))
    out1, out2 = pl.pallas_call(
        _exchange_kernel,
        grid=(B, GRID_C),
        in_specs=[mask_spec, mask_spec, x_spec, x_spec],
        out_specs=[x_spec, x_spec],
        out_shape=[
            jax.ShapeDtypeStruct((B, C, H, W), jnp.float32),
            jax.ShapeDtypeStruct((B, C, H, W), jnp.float32),
        ],
    )(m1, m2, x0, x1)
    return (out1, out2)


# speedup vs baseline: 8.6061x; 6.9354x over previous
import jax
import jax.numpy as jnp
from jax import lax
from jax.experimental import pallas as pl

C = 384
B, H, W = 16, 56, 56
HB = 28
GRID_H = H // HB


def _mask_kernel(w_ref, mask_ref):
    w = jnp.abs(w_ref[...])
    mn = jnp.min(w, axis=1, keepdims=True)
    mx = jnp.max(w, axis=1, keepdims=True)
    step = lax.broadcasted_iota(jnp.int32, (1, C), 1).astype(jnp.float32) / jnp.float32(C)
    e = mn * (1.0 - step) + mx * step
    cmp = (w[:, :, None] >= e[:, None, :]).astype(jnp.float32)
    cnt = jnp.sum(cmp, axis=1)
    zcol = jnp.zeros((2, 1), jnp.float32)
    cnt_next = jnp.concatenate([cnt[:, 1:], zcol], axis=1)
    hist = cnt - cnt_next
    hist_next = jnp.concatenate([hist[:, 1:], zcol], axis=1)
    d = hist_next - hist
    d_next = jnp.concatenate([d[:, 1:], zcol], axis=1)
    cond = (d <= 0.0) & (d_next > 0.0)
    idx = lax.broadcasted_iota(jnp.int32, (1, C), 1).astype(jnp.float32)
    valid = idx <= jnp.float32(C - 3)
    cand = jnp.where(cond & valid, idx, jnp.float32(1e9))
    first = jnp.min(cand, axis=1, keepdims=True)
    i_star = jnp.where(first > jnp.float32(C - 2), 0.0, first)
    thr = mn + ((i_star + 2.0) * (mx - mn)) / jnp.float32(C)
    mask_ref[...] = (w >= thr).astype(jnp.float32)


def _exchange_kernel(m_ref, x0_ref, x1_ref, o1_ref, o2_ref):
    m1 = m_ref[0:1, :][:, None, None, :] != 0.0
    m2 = m_ref[1:2, :][:, None, None, :] != 0.0
    x0 = x0_ref[...]
    x1 = x1_ref[...]
    o1_ref[...] = jnp.where(m1, x0, x1)
    o2_ref[...] = jnp.where(m2, x1, x0)


def kernel(x0, x1, bn1_weight, bn2_weight):
    wstack = jnp.stack([bn1_weight, bn2_weight])
    masks = pl.pallas_call(
        _mask_kernel,
        out_shape=jax.ShapeDtypeStruct((2, C), jnp.float32),
    )(wstack)

    x0t = jnp.transpose(x0, (0, 2, 3, 1))
    x1t = jnp.transpose(x1, (0, 2, 3, 1))

    x_spec = pl.BlockSpec((1, HB, W, C), lambda b, h: (b, h, 0, 0))
    mask_spec = pl.BlockSpec((2, C), lambda b, h: (0, 0))
    out1, out2 = pl.pallas_call(
        _exchange_kernel,
        grid=(B, GRID_H),
        in_specs=[mask_spec, x_spec, x_spec],
        out_specs=[x_spec, x_spec],
        out_shape=[
            jax.ShapeDtypeStruct((B, H, W, C), jnp.float32),
            jax.ShapeDtypeStruct((B, H, W, C), jnp.float32),
        ],
    )(masks, x0t, x1t)
    return (
        jnp.transpose(out1, (0, 3, 1, 2)),
        jnp.transpose(out2, (0, 3, 1, 2)),
    )
